# baseline (speedup 1.0000x reference)
import jax
import jax.numpy as jnp
from jax.experimental import pallas as pl
from jax.experimental.pallas import tpu as pltpu

B = 64
C = 16
K = 1024
E = 256
CW = C * E
Z = 512


def _leaky(v):
    return jnp.where(v >= 0, v, 0.2 * v)


def _mm(a, b):
    return jax.lax.dot_general(a, b, (((1,), (0,)), ((), ())),
                               preferred_element_type=jnp.float32)


def _fwd(xr_ref, bookT_ref, b2_ref,
         we1_ref, be1_ref, we2_ref, be2_ref, wef_ref, bef_ref,
         wi1_ref, bi1_ref, wp1_ref, bp1_ref, wp2_ref, bp2_ref,
         wq1z_ref, wq1h_ref, bq1_ref, wq2_ref, bq2_ref,
         wd1_ref, bd1_ref, wd2_ref, bd2_ref,
         cw_ref, dist_ref, idx_ref, mu_ref, lv_ref, plv_ref,
         dmu_ref, dlv_ref):
    h1 = _leaky(_mm(xr_ref[...], we1_ref[...]) + be1_ref[...])
    h2 = _leaky(_mm(h1, we2_ref[...]) + be2_ref[...])
    hp = h2[0:B]
    for c in range(1, C):
        hp = jnp.maximum(hp, h2[c * B:(c + 1) * B])
    h = _mm(hp, wef_ref[...]) + bef_ref[...]
    i1 = _mm(h, wi1_ref[...]) + bi1_ref[...]
    mu = i1[:, :Z // 4]
    mu_ref[...] = mu
    lv_ref[...] = i1[:, Z // 4:]
    p = _mm(_leaky(_mm(mu, wp1_ref[...]) + bp1_ref[...]),
            wp2_ref[...]) + bp2_ref[...]
    p_mu = p[:, :3 * Z // 4]
    plv_ref[...] = p[:, 3 * Z // 4:]
    qh = _leaky(_mm(mu, wq1z_ref[...]) + _mm(h, wq1h_ref[...]) + bq1_ref[...])
    q = _mm(qh, wq2_ref[...]) + bq2_ref[...]
    d_mu = q[:, :3 * Z // 4]
    dmu_ref[...] = d_mu
    dlv_ref[...] = q[:, 3 * Z // 4:]
    z2 = d_mu + p_mu
    d1 = _leaky(_mm(z2, wd1_ref[...]) + bd1_ref[...])
    cw = _mm(d1, wd2_ref[...]) + bd2_ref[...]
    cw_ref[...] = cw
    iota = jax.lax.broadcasted_iota(jnp.int32, (B, K), 1)
    for c in range(C):
        xc = cw[:, c * E:(c + 1) * E]
        x2 = jnp.sum(xc * xc, axis=1, keepdims=True)
        xb = _mm(xc, bookT_ref[c])
        dist = x2 - 2.0 * xb + b2_ref[c:c + 1, :]
        dist_ref[:, c * K:(c + 1) * K] = dist
        mn = jnp.min(dist, axis=1, keepdims=True)
        idx_ref[:, c:c + 1] = jnp.min(
            jnp.where(dist == mn, iota, K), axis=1, keepdims=True)


def kernel(x, codebook, W_e1, b_e1, W_e2, b_e2, W_ef, b_ef, W_i1, b_i1,
           W_p1, b_p1, W_p2, b_p2, W_q1, b_q1, W_q2, b_q2, W_d1, b_d1,
           W_d2, b_d2):
    f32 = jnp.float32
    xr = x.reshape(B, C, E).transpose(1, 0, 2).reshape(C * B, E)
    bookT = codebook.transpose(0, 2, 1)
    b2 = jnp.sum(codebook ** 2, axis=-1)
    args = (
        xr, bookT, b2,
        W_e1.T, b_e1.reshape(1, -1), W_e2.T, b_e2.reshape(1, -1),
        W_ef.T, b_ef.reshape(1, -1),
        W_i1.T, b_i1.reshape(1, -1),
        W_p1.T, b_p1.reshape(1, -1), W_p2.T, b_p2.reshape(1, -1),
        W_q1.T[:Z // 4, :], W_q1.T[Z // 4:, :], b_q1.reshape(1, -1),
        W_q2.T, b_q2.reshape(1, -1),
        W_d1.T, b_d1.reshape(1, -1), W_d2.T, b_d2.reshape(1, -1),
    )
    out_shape = [
        jax.ShapeDtypeStruct((B, CW), f32),
        jax.ShapeDtypeStruct((B, C * K), f32),
        jax.ShapeDtypeStruct((B, C), jnp.int32),
        jax.ShapeDtypeStruct((B, Z // 4), f32),
        jax.ShapeDtypeStruct((B, Z // 4), f32),
        jax.ShapeDtypeStruct((B, 3 * Z // 4), f32),
        jax.ShapeDtypeStruct((B, 3 * Z // 4), f32),
        jax.ShapeDtypeStruct((B, 3 * Z // 4), f32),
    ]
    cw, dist, idx, mu, lv, plv, dmu, dlv = pl.pallas_call(
        _fwd,
        out_shape=out_shape,
        compiler_params=pltpu.CompilerParams(
            vmem_limit_bytes=100 * 1024 * 1024),
    )(*args)
    return (cw, dist.reshape(B, C, K), idx.reshape(-1, 1), mu, lv,
            plv, dmu, dlv)


# speedup vs baseline: 2.1114x; 2.1114x over previous
import jax
import jax.numpy as jnp
from jax.experimental import pallas as pl
from jax.experimental.pallas import tpu as pltpu

B = 64
C = 16
K = 1024
E = 256
CW = C * E
Z = 512


def _leaky(v):
    return jnp.where(v >= 0, v, 0.2 * v)


def _mm(a, b):
    return jax.lax.dot_general(a, b, (((1,), (1,)), ((), ())),
                               preferred_element_type=jnp.float32)


def _fwd(xr_ref, book_ref, b2_ref,
         we1_ref, be1_ref, we2_ref, be2_ref, wef_ref, bef_ref,
         wi1_ref, bi1_ref, wp1_ref, bp1_ref, wp2_ref, bp2_ref,
         wq1_ref, bq1_ref, wq2_ref, bq2_ref,
         wd1_ref, bd1_ref, wd2_ref, bd2_ref,
         cw_ref, dist_ref, idx_ref, mu_ref, lv_ref, plv_ref,
         dmu_ref, dlv_ref):
    h1 = _leaky(_mm(xr_ref[...], we1_ref[...]) + be1_ref[...])
    h2 = _leaky(_mm(h1, we2_ref[...]) + be2_ref[...])
    hp = h2[0:B]
    for c in range(1, C):
        hp = jnp.maximum(hp, h2[c * B:(c + 1) * B])
    h = _mm(hp, wef_ref[...]) + bef_ref[...]
    i1 = _mm(h, wi1_ref[...]) + bi1_ref[...]
    mu = i1[:, :Z // 4]
    mu_ref[...] = mu
    lv_ref[...] = i1[:, Z // 4:]
    p = _mm(_leaky(_mm(mu, wp1_ref[...]) + bp1_ref[...]),
            wp2_ref[...]) + bp2_ref[...]
    p_mu = p[:, :3 * Z // 4]
    plv_ref[...] = p[:, 3 * Z // 4:]
    qh = _leaky(_mm(jnp.concatenate([mu, h], axis=1), wq1_ref[...])
                + bq1_ref[...])
    q = _mm(qh, wq2_ref[...]) + bq2_ref[...]
    d_mu = q[:, :3 * Z // 4]
    dmu_ref[...] = d_mu
    dlv_ref[...] = q[:, 3 * Z // 4:]
    z2 = d_mu + p_mu
    d1 = _leaky(_mm(z2, wd1_ref[...]) + bd1_ref[...])
    cw = _mm(d1, wd2_ref[...]) + bd2_ref[...]
    cw_ref[...] = cw
    iota = jax.lax.broadcasted_iota(jnp.int32, (B, K), 1)
    for c in range(C):
        xc = cw[:, c * E:(c + 1) * E]
        x2 = jnp.sum(xc * xc, axis=1, keepdims=True)
        xb = _mm(xc, book_ref[c])
        dist = x2 - 2.0 * xb + b2_ref[c:c + 1, :]
        dist_ref[:, c * K:(c + 1) * K] = dist
        mn = jnp.min(dist, axis=1, keepdims=True)
        idx_ref[:, c:c + 1] = jnp.min(
            jnp.where(dist == mn, iota, K), axis=1, keepdims=True)


def kernel(x, codebook, W_e1, b_e1, W_e2, b_e2, W_ef, b_ef, W_i1, b_i1,
           W_p1, b_p1, W_p2, b_p2, W_q1, b_q1, W_q2, b_q2, W_d1, b_d1,
           W_d2, b_d2):
    f32 = jnp.float32
    xr = x.reshape(B, C, E).transpose(1, 0, 2).reshape(C * B, E)
    b2 = jnp.sum(codebook ** 2, axis=-1)
    args = (
        xr, codebook, b2,
        W_e1, b_e1.reshape(1, -1), W_e2, b_e2.reshape(1, -1),
        W_ef, b_ef.reshape(1, -1),
        W_i1, b_i1.reshape(1, -1),
        W_p1, b_p1.reshape(1, -1), W_p2, b_p2.reshape(1, -1),
        W_q1, b_q1.reshape(1, -1),
        W_q2, b_q2.reshape(1, -1),
        W_d1, b_d1.reshape(1, -1), W_d2, b_d2.reshape(1, -1),
    )
    out_shape = [
        jax.ShapeDtypeStruct((B, CW), f32),
        jax.ShapeDtypeStruct((B, C * K), f32),
        jax.ShapeDtypeStruct((B, C), jnp.int32),
        jax.ShapeDtypeStruct((B, Z // 4), f32),
        jax.ShapeDtypeStruct((B, Z // 4), f32),
        jax.ShapeDtypeStruct((B, 3 * Z // 4), f32),
        jax.ShapeDtypeStruct((B, 3 * Z // 4), f32),
        jax.ShapeDtypeStruct((B, 3 * Z // 4), f32),
    ]
    cw, dist, idx, mu, lv, plv, dmu, dlv = pl.pallas_call(
        _fwd,
        out_shape=out_shape,
        compiler_params=pltpu.CompilerParams(
            vmem_limit_bytes=100 * 1024 * 1024),
    )(*args)
    return (cw, dist.reshape(B, C, K), idx.reshape(-1, 1), mu, lv,
            plv, dmu, dlv)
